# initial kernel scaffold (unmeasured)
import jax
import jax.numpy as jnp
from jax import lax
from jax.experimental import pallas as pl
from jax.experimental.pallas import tpu as pltpu

N_DEV = 32
M = 4096
N = 8192
CHUNK = N // N_DEV


def _gelu(y):
    c = 0.7978845608028654
    return 0.5 * y * (1.0 + jnp.tanh(c * (y + 0.044715 * y * y * y)))


def _ar_body(p_ref, out_ref, send_ref, recv_ref, local_ref,
             local_sem, out_sem, send_sems, recv_sems,
             credit0, credit1):
    d = lax.axis_index("i")
    left = lax.rem(d - 1 + N_DEV, N_DEV)
    right = lax.rem(d + 1, N_DEV)

    barrier_sem = pltpu.get_barrier_semaphore()
    for nbr in (left, right):
        pl.semaphore_signal(barrier_sem, inc=1, device_id=(nbr,),
                            device_id_type=pl.DeviceIdType.MESH)
    pl.semaphore_wait(barrier_sem, 2)

    credits = (credit0, credit1)

    def col(chunk_idx):
        return pl.ds(chunk_idx * CHUNK, CHUNK)

    cp = pltpu.make_async_copy(p_ref.at[:, col(d)], send_ref, local_sem)
    cp.start()
    cp.wait()

    for s in range(N_DEV - 1):
        g = s
        slot = g % 2
        if g >= 2:
            pl.semaphore_wait(credits[slot], 1)
        rdma = pltpu.make_async_remote_copy(
            src_ref=send_ref,
            dst_ref=recv_ref.at[slot],
            send_sem=send_sems.at[slot],
            recv_sem=recv_sems.at[slot],
            device_id=(right,),
            device_id_type=pl.DeviceIdType.MESH,
        )
        rdma.start()
        rc = lax.rem(d - s - 1 + N_DEV, N_DEV)
        lcp = pltpu.make_async_copy(p_ref.at[:, col(rc)], local_ref, local_sem)
        lcp.start()
        rdma.wait_send()
        rdma.wait_recv()
        lcp.wait()
        send_ref[...] = recv_ref[slot] + local_ref[...]
        pl.semaphore_signal(credits[slot], inc=1, device_id=(left,),
                            device_id_type=pl.DeviceIdType.MESH)

    own = lax.rem(d + 1, N_DEV)
    send_ref[...] = _gelu(send_ref[...])
    ocp = pltpu.make_async_copy(send_ref, out_ref.at[:, col(own)], out_sem)
    ocp.start()
    ocp.wait()

    for t in range(N_DEV - 1):
        g = (N_DEV - 1) + t
        slot = g % 2
        if g >= 2:
            pl.semaphore_wait(credits[slot], 1)
        rdma = pltpu.make_async_remote_copy(
            src_ref=send_ref,
            dst_ref=recv_ref.at[slot],
            send_sem=send_sems.at[slot],
            recv_sem=recv_sems.at[slot],
            device_id=(right,),
            device_id_type=pl.DeviceIdType.MESH,
        )
        rdma.start()
        rdma.wait_send()
        rdma.wait_recv()
        rc = lax.rem(d - t + N_DEV, N_DEV)
        send_ref[...] = recv_ref[slot]
        ocp = pltpu.make_async_copy(send_ref, out_ref.at[:, col(rc)], out_sem)
        ocp.start()
        ocp.wait()
        pl.semaphore_signal(credits[slot], inc=1, device_id=(left,),
                            device_id_type=pl.DeviceIdType.MESH)


def kernel(x, w_mat):
    partial = jnp.dot(x, w_mat, preferred_element_type=jnp.float32)
    return pl.pallas_call(
        _ar_body,
        out_shape=jax.ShapeDtypeStruct((M, N), jnp.float32),
        in_specs=[pl.BlockSpec(memory_space=pltpu.HBM)],
        out_specs=pl.BlockSpec(memory_space=pltpu.HBM),
        scratch_shapes=[
            pltpu.VMEM((M, CHUNK), jnp.float32),
            pltpu.VMEM((2, M, CHUNK), jnp.float32),
            pltpu.VMEM((M, CHUNK), jnp.float32),
            pltpu.SemaphoreType.DMA,
            pltpu.SemaphoreType.DMA,
            pltpu.SemaphoreType.DMA((2,)),
            pltpu.SemaphoreType.DMA((2,)),
            pltpu.SemaphoreType.REGULAR,
            pltpu.SemaphoreType.REGULAR,
        ],
        compiler_params=pltpu.CompilerParams(collective_id=0),
    )(partial)


# baseline (device time: 3170619 ns/iter reference)
import functools

import jax
import jax.numpy as jnp
from jax import lax
from jax.experimental import pallas as pl
from jax.experimental.pallas import tpu as pltpu

N_DEV = 32
M = 4096
N = 8192


def _gelu(y):
    c = 0.7978845608028654
    return 0.5 * y * (1.0 + jnp.tanh(c * (y + 0.044715 * y * y * y)))


def _ar_body(p_ref, out_ref, send_ref, recv_ref, local_ref, dummy_ref,
             local_sem, out_sem, send_sems, recv_sems,
             crd_send_sems, crd_recv_sems, *, chunk, apply_gelu=True):
    CHUNK = chunk
    n_steps = 2 * (N_DEV - 1)
    d = lax.axis_index("i")
    left = lax.rem(d - 1 + N_DEV, N_DEV)
    right = lax.rem(d + 1, N_DEV)

    barrier_sem = pltpu.get_barrier_semaphore()
    for nbr in (left, right):
        pl.semaphore_signal(barrier_sem, inc=1, device_id=(nbr,),
                            device_id_type=pl.DeviceIdType.MESH)
    pl.semaphore_wait(barrier_sem, 2)

    def col(chunk_idx):
        return pl.ds(chunk_idx * CHUNK, CHUNK)

    def credit_wait(slot):
        crd = pltpu.make_async_remote_copy(
            src_ref=dummy_ref, dst_ref=dummy_ref,
            send_sem=crd_send_sems.at[slot], recv_sem=crd_recv_sems.at[slot],
            device_id=(left,), device_id_type=pl.DeviceIdType.MESH)
        crd.wait_recv()

    def credit_send(slot):
        crd = pltpu.make_async_remote_copy(
            src_ref=dummy_ref, dst_ref=dummy_ref,
            send_sem=crd_send_sems.at[slot], recv_sem=crd_recv_sems.at[slot],
            device_id=(left,), device_id_type=pl.DeviceIdType.MESH)
        crd.start()
        crd.wait_send()

    cp = pltpu.make_async_copy(p_ref.at[:, col(d)], send_ref, local_sem)
    cp.start()
    cp.wait()

    for s in range(N_DEV - 1):
        g = s
        slot = g % 2
        if g >= 2:
            credit_wait(slot)
        rdma = pltpu.make_async_remote_copy(
            src_ref=send_ref,
            dst_ref=recv_ref.at[slot],
            send_sem=send_sems.at[slot],
            recv_sem=recv_sems.at[slot],
            device_id=(right,),
            device_id_type=pl.DeviceIdType.MESH,
        )
        rdma.start()
        rc = lax.rem(d - s - 1 + N_DEV, N_DEV)
        lcp = pltpu.make_async_copy(p_ref.at[:, col(rc)], local_ref, local_sem)
        lcp.start()
        rdma.wait_send()
        rdma.wait_recv()
        lcp.wait()
        send_ref[...] = recv_ref[slot] + local_ref[...]
        if g <= n_steps - 3:
            credit_send(slot)

    own = lax.rem(d + 1, N_DEV)
    if apply_gelu:
        send_ref[...] = _gelu(send_ref[...])
    ocp = pltpu.make_async_copy(send_ref, out_ref.at[:, col(own)], out_sem)
    ocp.start()
    ocp.wait()

    for t in range(N_DEV - 1):
        g = (N_DEV - 1) + t
        slot = g % 2
        if g >= 2:
            credit_wait(slot)
        rdma = pltpu.make_async_remote_copy(
            src_ref=send_ref,
            dst_ref=recv_ref.at[slot],
            send_sem=send_sems.at[slot],
            recv_sem=recv_sems.at[slot],
            device_id=(right,),
            device_id_type=pl.DeviceIdType.MESH,
        )
        rdma.start()
        rdma.wait_send()
        rdma.wait_recv()
        rc = lax.rem(d - t + N_DEV, N_DEV)
        send_ref[...] = recv_ref[slot]
        ocp = pltpu.make_async_copy(send_ref, out_ref.at[:, col(rc)], out_sem)
        ocp.start()
        ocp.wait()
        if g <= n_steps - 3:
            credit_send(slot)


def ring_allreduce(partial, *, apply_gelu=True):
    m, n = partial.shape
    chunk = n // N_DEV
    body = functools.partial(_ar_body, chunk=chunk, apply_gelu=apply_gelu)
    return pl.pallas_call(
        body,
        out_shape=jax.ShapeDtypeStruct((m, n), jnp.float32),
        in_specs=[pl.BlockSpec(memory_space=pltpu.HBM)],
        out_specs=pl.BlockSpec(memory_space=pltpu.HBM),
        scratch_shapes=[
            pltpu.VMEM((m, chunk), jnp.float32),
            pltpu.VMEM((2, m, chunk), jnp.float32),
            pltpu.VMEM((m, chunk), jnp.float32),
            pltpu.VMEM((8, 128), jnp.float32),
            pltpu.SemaphoreType.DMA,
            pltpu.SemaphoreType.DMA,
            pltpu.SemaphoreType.DMA((2,)),
            pltpu.SemaphoreType.DMA((2,)),
            pltpu.SemaphoreType.DMA((2,)),
            pltpu.SemaphoreType.DMA((2,)),
        ],
        compiler_params=pltpu.CompilerParams(collective_id=0),
    )(partial)


def kernel(x, w_mat):
    partial = jnp.dot(x, w_mat, preferred_element_type=jnp.float32)
    return ring_allreduce(partial, apply_gelu=True)


# device time: 1711341 ns/iter; 1.8527x vs baseline; 1.8527x over previous
import functools

import jax
import jax.numpy as jnp
from jax import lax
from jax.experimental import pallas as pl
from jax.experimental.pallas import tpu as pltpu

N_DEV = 32
M = 4096
N = 8192


def _gelu(y):
    c = 0.7978845608028654
    return 0.5 * y * (1.0 + jnp.tanh(c * (y + 0.044715 * y * y * y)))


def _ar_body(p_ref, out_ref,
             sendA, sendB, recvA, recvB, localA, localB, stageA, stageB,
             lsemA, lsemB, osemA, osemB,
             ssemsA, rsemsA, csendA, crecvA,
             ssemsB, rsemsB, csendB, crecvB,
             dummy,
             *, chunk, rows_half, apply_gelu=True):
    CHUNK = chunk
    RH = rows_half
    n_steps = 2 * (N_DEV - 1)
    d = lax.axis_index("i")
    left = lax.rem(d - 1 + N_DEV, N_DEV)
    right = lax.rem(d + 1, N_DEV)

    barrier_sem = pltpu.get_barrier_semaphore()
    for nbr in (left, right):
        pl.semaphore_signal(barrier_sem, inc=1, device_id=(nbr,),
                            device_id_type=pl.DeviceIdType.MESH)
    pl.semaphore_wait(barrier_sem, 2)

    def col(chunk_idx):
        return pl.ds(chunk_idx * CHUNK, CHUNK)

    def credit(sems_pair, slot, to):
        csend, crecv = sems_pair
        return pltpu.make_async_remote_copy(
            src_ref=dummy, dst_ref=dummy,
            send_sem=csend.at[slot], recv_sem=crecv.at[slot],
            device_id=(to,), device_id_type=pl.DeviceIdType.MESH)

    def data_rdma(ring, slot):
        if ring == 0:
            return pltpu.make_async_remote_copy(
                src_ref=sendA, dst_ref=recvA.at[slot],
                send_sem=ssemsA.at[slot], recv_sem=rsemsA.at[slot],
                device_id=(right,), device_id_type=pl.DeviceIdType.MESH)
        return pltpu.make_async_remote_copy(
            src_ref=sendB, dst_ref=recvB.at[slot],
            send_sem=ssemsB.at[slot], recv_sem=rsemsB.at[slot],
            device_id=(left,), device_id_type=pl.DeviceIdType.MESH)

    rows = (pl.ds(0, RH), pl.ds(RH, RH))

    cpA = pltpu.make_async_copy(p_ref.at[rows[0], col(d)], localA, lsemA)
    cpB = pltpu.make_async_copy(p_ref.at[rows[1], col(d)], localB, lsemB)
    cpA.start()
    cpB.start()
    cpA.wait()
    cpB.wait()
    sendA[...] = localA[...].astype(jnp.bfloat16)
    sendB[...] = localB[...].astype(jnp.bfloat16)

    for s in range(N_DEV - 1):
        g = s
        slot = g % 2
        if g >= 2:
            credit((csendA, crecvA), slot, left).wait_recv()
            credit((csendB, crecvB), slot, right).wait_recv()
        rdA = data_rdma(0, slot)
        rdB = data_rdma(1, slot)
        rdA.start()
        rdB.start()
        rcA = lax.rem(d - s - 1 + N_DEV, N_DEV)
        rcB = lax.rem(d + s + 1, N_DEV)
        lcpA = pltpu.make_async_copy(p_ref.at[rows[0], col(rcA)], localA, lsemA)
        lcpB = pltpu.make_async_copy(p_ref.at[rows[1], col(rcB)], localB, lsemB)
        lcpA.start()
        lcpB.start()
        rdA.wait_send()
        rdB.wait_send()
        rdA.wait_recv()
        rdB.wait_recv()
        lcpA.wait()
        lcpB.wait()
        sendA[...] = (recvA[slot].astype(jnp.float32)
                      + localA[...]).astype(jnp.bfloat16)
        sendB[...] = (recvB[slot].astype(jnp.float32)
                      + localB[...]).astype(jnp.bfloat16)
        if g <= n_steps - 3:
            crA = credit((csendA, crecvA), slot, left)
            crB = credit((csendB, crecvB), slot, right)
            crA.start()
            crB.start()
            crA.wait_send()
            crB.wait_send()

    ownA = lax.rem(d + 1, N_DEV)
    ownB = lax.rem(d - 1 + N_DEV, N_DEV)
    accA = sendA[...].astype(jnp.float32)
    accB = sendB[...].astype(jnp.float32)
    if apply_gelu:
        accA = _gelu(accA)
        accB = _gelu(accB)
    stageA[...] = accA
    stageB[...] = accB
    sendA[...] = accA.astype(jnp.bfloat16)
    sendB[...] = accB.astype(jnp.bfloat16)
    ocpA = pltpu.make_async_copy(stageA, out_ref.at[rows[0], col(ownA)], osemA)
    ocpB = pltpu.make_async_copy(stageB, out_ref.at[rows[1], col(ownB)], osemB)
    ocpA.start()
    ocpB.start()

    for t in range(N_DEV - 1):
        g = (N_DEV - 1) + t
        slot = g % 2
        if g >= 2:
            credit((csendA, crecvA), slot, left).wait_recv()
            credit((csendB, crecvB), slot, right).wait_recv()
        rdA = data_rdma(0, slot)
        rdB = data_rdma(1, slot)
        rdA.start()
        rdB.start()
        rcA = lax.rem(d - t + N_DEV, N_DEV)
        rcB = lax.rem(d + t, N_DEV)
        rdA.wait_send()
        rdB.wait_send()
        rdA.wait_recv()
        rdB.wait_recv()
        ocpA.wait()
        ocpB.wait()
        stageA[...] = recvA[slot].astype(jnp.float32)
        stageB[...] = recvB[slot].astype(jnp.float32)
        sendA[...] = recvA[slot]
        sendB[...] = recvB[slot]
        ocpA = pltpu.make_async_copy(stageA, out_ref.at[rows[0], col(rcA)], osemA)
        ocpB = pltpu.make_async_copy(stageB, out_ref.at[rows[1], col(rcB)], osemB)
        ocpA.start()
        ocpB.start()
        if g <= n_steps - 3:
            crA = credit((csendA, crecvA), slot, left)
            crB = credit((csendB, crecvB), slot, right)
            crA.start()
            crB.start()
            crA.wait_send()
            crB.wait_send()
    ocpA.wait()
    ocpB.wait()


def ring_allreduce(partial, *, apply_gelu=True):
    m, n = partial.shape
    chunk = n // N_DEV
    rh = m // 2
    body = functools.partial(_ar_body, chunk=chunk, rows_half=rh,
                             apply_gelu=apply_gelu)
    bf = jnp.bfloat16
    f32 = jnp.float32
    return pl.pallas_call(
        body,
        out_shape=jax.ShapeDtypeStruct((m, n), f32),
        in_specs=[pl.BlockSpec(memory_space=pltpu.HBM)],
        out_specs=pl.BlockSpec(memory_space=pltpu.HBM),
        scratch_shapes=[
            pltpu.VMEM((rh, chunk), bf),
            pltpu.VMEM((rh, chunk), bf),
            pltpu.VMEM((2, rh, chunk), bf),
            pltpu.VMEM((2, rh, chunk), bf),
            pltpu.VMEM((rh, chunk), f32),
            pltpu.VMEM((rh, chunk), f32),
            pltpu.VMEM((rh, chunk), f32),
            pltpu.VMEM((rh, chunk), f32),
            pltpu.SemaphoreType.DMA,
            pltpu.SemaphoreType.DMA,
            pltpu.SemaphoreType.DMA,
            pltpu.SemaphoreType.DMA,
            pltpu.SemaphoreType.DMA((2,)),
            pltpu.SemaphoreType.DMA((2,)),
            pltpu.SemaphoreType.DMA((2,)),
            pltpu.SemaphoreType.DMA((2,)),
            pltpu.SemaphoreType.DMA((2,)),
            pltpu.SemaphoreType.DMA((2,)),
            pltpu.SemaphoreType.DMA((2,)),
            pltpu.SemaphoreType.DMA((2,)),
            pltpu.VMEM((8, 128), f32),
        ],
        compiler_params=pltpu.CompilerParams(collective_id=0),
    )(partial)


def kernel(x, w_mat):
    partial = jnp.dot(x, w_mat, preferred_element_type=jnp.float32)
    return ring_allreduce(partial, apply_gelu=True)
